# Initial kernel scaffold; baseline (speedup 1.0000x reference)
import jax
import jax.numpy as jnp
from jax.experimental import pallas as pl


def kernel(x, filters, bias):
    raise NotImplementedError("write your pallas kernel here")


# speedup vs baseline: 1.4618x; 1.4618x over previous
import jax
import jax.numpy as jnp
from jax.experimental import pallas as pl
from jax.experimental.pallas import tpu as pltpu

_B, _C, _H, _W = 8, 3, 512, 512
_D, _K = 64, 3
_OH, _OW = _H - _K + 1, _W - _K + 1
_TH = 51
_NH = _OH // _TH


def _conv_body(x_ref, w_ref, b_ref, o_ref):
    slabs = []
    for c in range(_C):
        v = x_ref[0, c, 0]
        for dy in range(_K):
            for dx in range(_K):
                slabs.append(v[dy:dy + _TH, dx:dx + _OW])
    patch = jnp.stack(slabs, axis=0)
    patch = jnp.concatenate(
        [patch, jnp.ones((1, _TH, _OW), jnp.float32)], axis=0)
    bsum = jnp.sum(b_ref[...], axis=1, keepdims=True)
    w_aug = jnp.concatenate([w_ref[...], bsum], axis=1)
    o_ref[0, :, 0] = jnp.einsum(
        "dk,ktj->dtj", w_aug, patch,
        preferred_element_type=jnp.float32,
    )


def kernel(x, filters, bias):
    w2 = filters.reshape(_D, _C * _K * _K)
    b2 = bias.reshape(_D, _C * _K * _K)
    x_t = jnp.stack(
        [x[:, :, _TH * i:_TH * i + _TH + 2, :] for i in range(_NH)], axis=2
    )
    out = pl.pallas_call(
        _conv_body,
        grid=(_B, _NH),
        in_specs=[
            pl.BlockSpec((1, _C, 1, _TH + 2, _W), lambda b, i: (b, 0, i, 0, 0)),
            pl.BlockSpec((_D, _C * _K * _K), lambda b, i: (0, 0)),
            pl.BlockSpec((_D, _C * _K * _K), lambda b, i: (0, 0)),
        ],
        out_specs=pl.BlockSpec((1, _D, 1, _TH, _OW), lambda b, i: (b, 0, i, 0, 0)),
        out_shape=jax.ShapeDtypeStruct((_B, _D, _NH, _TH, _OW), jnp.float32),
        compiler_params=pltpu.CompilerParams(
            dimension_semantics=("parallel", "arbitrary"),
        ),
    )(x_t, w2, b2)
    return out.reshape(_B, _D, _OH, _OW)


# speedup vs baseline: 2.0697x; 1.4159x over previous
import jax
import jax.numpy as jnp
from jax.experimental import pallas as pl
from jax.experimental.pallas import tpu as pltpu

_B, _C, _H, _W = 8, 3, 512, 512
_D, _K = 64, 3
_OH, _OW = _H - _K + 1, _W - _K + 1
_TW = 128
_NW = (_OW + _TW - 1) // _TW


def _conv_body(xa_ref, xb_ref, w_ref, b_ref, o_ref):
    slabs = []
    for c in range(_C):
        full = jnp.concatenate([xa_ref[0, c], xb_ref[0, c]], axis=1)
        for dy in range(_K):
            for dx in range(_K):
                slabs.append(full[dy:dy + _OH, dx:dx + _TW])
    patch = jnp.stack(slabs, axis=0)
    patch = jnp.concatenate(
        [patch, jnp.ones((1, _OH, _TW), jnp.float32)], axis=0)
    bsum = jnp.sum(b_ref[...], axis=1, keepdims=True)
    w_aug = jnp.concatenate([w_ref[...], bsum], axis=1)
    o_ref[0] = jnp.einsum(
        "dk,ktj->dtj", w_aug, patch,
        preferred_element_type=jnp.float32,
    )


def kernel(x, filters, bias):
    w2 = filters.reshape(_D, _C * _K * _K)
    b2 = bias.reshape(_D, _C * _K * _K)
    return pl.pallas_call(
        _conv_body,
        grid=(_B, _NW),
        in_specs=[
            pl.BlockSpec((1, _C, _H, _TW), lambda b, j: (b, 0, 0, j)),
            pl.BlockSpec(
                (1, _C, _H, _TW),
                lambda b, j: (b, 0, 0, jnp.minimum(j + 1, _NW - 1))),
            pl.BlockSpec((_D, _C * _K * _K), lambda b, j: (0, 0)),
            pl.BlockSpec((_D, _C * _K * _K), lambda b, j: (0, 0)),
        ],
        out_specs=pl.BlockSpec((1, _D, _OH, _TW), lambda b, j: (b, 0, 0, j)),
        out_shape=jax.ShapeDtypeStruct((_B, _D, _OH, _OW), jnp.float32),
        compiler_params=pltpu.CompilerParams(
            dimension_semantics=("parallel", "arbitrary"),
        ),
    )(x, x, w2, b2)
